# baseline (speedup 1.0000x reference)
import jax
import jax.numpy as jnp
from jax import lax
from jax.experimental import pallas as pl

_TOK_BLK = 2048


def _router_body(h_ref, w_ref, b_ref, logits_ref, idx_ref):
    h_blk = h_ref[...]
    w = w_ref[...]
    logits = lax.dot_general(h_blk, w, (((1,), (1,)), ((), ())),
                             preferred_element_type=jnp.float32)
    logits = logits + b_ref[...]
    logits_ref[...] = logits
    rowmax = jnp.max(logits, axis=1, keepdims=True)
    eidx = lax.broadcasted_iota(jnp.int32, logits.shape, 1)
    masked = jnp.where(logits == rowmax, eidx, logits.shape[1])
    idx_ref[...] = jnp.min(masked, axis=1)


def kernel(h, W, b):
    n, d = h.shape
    e = W.shape[0]
    logits, idx = pl.pallas_call(
        _router_body,
        grid=(n // _TOK_BLK,),
        in_specs=[
            pl.BlockSpec((_TOK_BLK, d), lambda i: (i, 0)),
            pl.BlockSpec((e, d), lambda i: (0, 0)),
            pl.BlockSpec((1, e), lambda i: (0, 0)),
        ],
        out_specs=[
            pl.BlockSpec((_TOK_BLK, e), lambda i: (i, 0)),
            pl.BlockSpec((_TOK_BLK,), lambda i: (i,)),
        ],
        out_shape=[
            jax.ShapeDtypeStruct((n, e), jnp.float32),
            jax.ShapeDtypeStruct((n,), jnp.int32),
        ],
    )(h, W, b.reshape(1, e))
    return (logits, idx)


# speedup vs baseline: 1.7175x; 1.7175x over previous
import jax
import jax.numpy as jnp
from jax import lax
from jax.experimental import pallas as pl

_TOK_BLK = 2048


def _router_body(h_ref, w_ref, b_ref, logits_ref, idx_ref):
    h_blk = h_ref[...]
    w = w_ref[...]
    logits = lax.dot_general(h_blk, w, (((1,), (1,)), ((), ())),
                             preferred_element_type=jnp.float32)
    logits = logits + b_ref[...]
    logits_ref[...] = logits
    lt = logits.T
    colmax = jnp.max(lt, axis=0, keepdims=True)
    eidx = lax.broadcasted_iota(jnp.int32, lt.shape, 0)
    masked = jnp.where(lt == colmax, eidx, lt.shape[0])
    idx_ref[...] = jnp.min(masked, axis=0)


def kernel(h, W, b):
    n, d = h.shape
    e = W.shape[0]
    logits, idx = pl.pallas_call(
        _router_body,
        grid=(n // _TOK_BLK,),
        in_specs=[
            pl.BlockSpec((_TOK_BLK, d), lambda i: (i, 0)),
            pl.BlockSpec((e, d), lambda i: (0, 0)),
            pl.BlockSpec((1, e), lambda i: (0, 0)),
        ],
        out_specs=[
            pl.BlockSpec((_TOK_BLK, e), lambda i: (i, 0)),
            pl.BlockSpec((_TOK_BLK,), lambda i: (i,)),
        ],
        out_shape=[
            jax.ShapeDtypeStruct((n, e), jnp.float32),
            jax.ShapeDtypeStruct((n,), jnp.int32),
        ],
    )(h, W, b.reshape(1, e))
    return (logits, idx)


# speedup vs baseline: 1.7630x; 1.0265x over previous
import jax
import jax.numpy as jnp
from jax import lax
from jax.experimental import pallas as pl

_TOK_BLK = 4096


def _router_body(h_ref, w_ref, b_ref, logits_ref, idx_ref):
    h_blk = h_ref[...]
    w = w_ref[...]
    logits = lax.dot_general(h_blk, w, (((1,), (1,)), ((), ())),
                             preferred_element_type=jnp.float32)
    logits = logits + b_ref[...]
    logits_ref[...] = logits
    lt = logits.T
    colmax = jnp.max(lt, axis=0, keepdims=True)
    eidx = lax.broadcasted_iota(jnp.int32, lt.shape, 0)
    masked = jnp.where(lt == colmax, eidx, lt.shape[0])
    idx_ref[...] = jnp.min(masked, axis=0)


def kernel(h, W, b):
    n, d = h.shape
    e = W.shape[0]
    logits, idx = pl.pallas_call(
        _router_body,
        grid=(n // _TOK_BLK,),
        in_specs=[
            pl.BlockSpec((_TOK_BLK, d), lambda i: (i, 0)),
            pl.BlockSpec((e, d), lambda i: (0, 0)),
            pl.BlockSpec((1, e), lambda i: (0, 0)),
        ],
        out_specs=[
            pl.BlockSpec((_TOK_BLK, e), lambda i: (i, 0)),
            pl.BlockSpec((_TOK_BLK,), lambda i: (i,)),
        ],
        out_shape=[
            jax.ShapeDtypeStruct((n, e), jnp.float32),
            jax.ShapeDtypeStruct((n,), jnp.int32),
        ],
    )(h, W, b.reshape(1, e))
    return (logits, idx)
